# baseline (speedup 1.0000x reference)
import jax
import jax.numpy as jnp
from jax import lax
from jax.experimental import pallas as pl
from jax.experimental.pallas import tpu as pltpu
from jax.experimental.pallas import tpu_sc as plsc

_B = 16384
_D = 64
_NW = 32
_BPW = _B // _NW
_CHUNK = 128
_NCHUNK = _BPW // _CHUNK
_L = 16


_HALF = _BPW // 2
_HCHUNK = _HALF // _CHUNK


def _gather_kernel(x2_hbm, idx_hbm, out_hbm, idx_v, gidx_v, pairs_v, rows_v, sem):
    nc = 2
    wid = lax.axis_index("s") * nc + lax.axis_index("c")
    base = wid * _BPW
    pltpu.sync_copy(idx_hbm.at[pl.ds(base, _BPW)], idx_v)
    iota = lax.iota(jnp.int32, _L)

    for h in range(2):
        h0 = h * _HALF

        def fill(v, _):
            e = idx_v[pl.ds(h0 + v * _L, _L)]
            gidx_v[v // (_CHUNK // _L), pl.ds((v % (_CHUNK // _L)) * _L, _L)] = (
                lax.shift_right_logical(e, 1))
            return _

        lax.fori_loop(0, _HALF // _L, fill, 0, unroll=True)

        descs = [
            pltpu.async_copy(
                x2_hbm.at[gidx_v.at[j]],
                pairs_v.at[pl.ds(j * _CHUNK, _CHUNK)],
                sem,
            )
            for j in range(_HCHUNK)
        ]
        for d in descs:
            d.wait()

        def select(v, _):
            j_vec = v * _L + iota
            par = jnp.bitwise_and(idx_v[pl.ds(h0 + v * _L, _L)], 1)
            col0 = par * _D
            for f in range(_D):
                val = plsc.load_gather(pairs_v, [j_vec, col0 + f])
                plsc.store_scatter(
                    rows_v, [h0 + j_vec, jnp.full((_L,), f, jnp.int32)], val)
            return _

        lax.fori_loop(0, _HALF // _L, select, 0)

    pltpu.sync_copy(rows_v, out_hbm.at[pl.ds(base, _BPW)])


@jax.jit
def _gather(x2, idx):
    mesh = plsc.VectorSubcoreMesh(core_axis_name="c", subcore_axis_name="s")
    fn = pl.kernel(
        _gather_kernel,
        mesh=mesh,
        out_type=jax.ShapeDtypeStruct((_B, _D), jnp.float32),
        scratch_types=[
            pltpu.VMEM((_BPW,), jnp.int32),
            pltpu.VMEM((_HCHUNK, _CHUNK), jnp.int32),
            pltpu.VMEM((_HALF, 2 * _D), jnp.float32),
            pltpu.VMEM((_BPW, _D), jnp.float32),
            pltpu.SemaphoreType.DMA,
        ],
        compiler_params=pltpu.CompilerParams(
            use_tc_tiling_on_sc=True, needs_layout_passes=False),
    )
    return fn(x2, idx)


def kernel(x, indices, out):
    x2 = x.reshape(500000, 2 * _D)
    return _gather(x2, indices)


# speedup vs baseline: 2.7845x; 2.7845x over previous
import jax
import jax.numpy as jnp
from jax import lax
from jax.experimental import pallas as pl
from jax.experimental.pallas import tpu as pltpu
from jax.experimental.pallas import tpu_sc as plsc

_B = 16384
_D = 64
_NW = 32
_L = 16
_V = 1000000
_FULL_TCOLS = _V // 128
_TAIL_BASE = _FULL_TCOLS * 128
_VC = 248
_CB = 4
_BLK_LANES = _CB * 128
_RCAP = 128
_IDXCHUNK = 4096


def _gather_kernel(xt_hbm, idx_hbm, tail_hbm, out_hbm,
                   idxbuf, selp, blkp, buf, tailb, rows_v, bi_v,
                   sem_in, sem_out):
    nc = 2
    wid = lax.axis_index("s") * nc + lax.axis_index("c")
    iota = lax.iota(jnp.int32, _L)
    c_lo = wid * _VC
    c_hi_full = jnp.minimum(c_lo + _VC, _FULL_TCOLS)
    is_last = wid == (_NW - 1)
    c_hi_sel = jnp.where(is_last, _FULL_TCOLS + 1, c_hi_full)

    def sel_chunk(q, off):
        pltpu.sync_copy(idx_hbm.at[pl.ds(q * _IDXCHUNK, _IDXCHUNK)], idxbuf)

        def sel_v(v, off):
            e = idxbuf[pl.ds(v * _L, _L)]
            c = lax.shift_right_logical(e, 7)
            m = jnp.logical_and(c >= c_lo, c < c_hi_sel)
            i_g = q * _IDXCHUNK + v * _L + iota
            pk = jnp.bitwise_or(lax.shift_left(i_g, 15), e - c_lo * 128)
            plsc.store_compressed(selp.at[pl.ds(off, _L)], pk, mask=m)
            return off + jnp.sum(m.astype(jnp.int32))

        return lax.fori_loop(0, _IDXCHUNK // _L, sel_v, off)

    nsel = lax.fori_loop(0, _B // _IDXCHUNK, sel_chunk, 0)
    nv_sel = lax.shift_right_logical(nsel + _L - 1, 4)

    nb = lax.shift_right_logical(
        jnp.maximum(c_hi_full - c_lo, 0) + _CB - 1, 2)

    def compact_block(lo_r, hi_r):
        def cmp_v(v, boff):
            pk = selp[pl.ds(v * _L, _L)]
            valid = (v * _L + iota) < nsel
            cr = lax.shift_right_logical(jnp.bitwise_and(pk, 32767), 7)
            m = jnp.logical_and(valid,
                                jnp.logical_and(cr >= lo_r, cr < hi_r))
            plsc.store_compressed(blkp.at[pl.ds(boff, _L)], pk, mask=m)
            return boff + jnp.sum(m.astype(jnp.int32))

        return lax.fori_loop(0, nv_sel, cmp_v, 0)

    def stage_entries(v, R, boff, er_base, loader):
        pk = blkp[pl.ds(v * _L, _L)]
        valid = (v * _L + iota) < boff
        pkm = jnp.where(valid, pk, jnp.int32(1 << 30))
        mn = jnp.min(pkm, axis=0)
        pku = jnp.where(valid, pk, jnp.full((_L,), 1, jnp.int32) * mn)
        er = jnp.bitwise_and(pku, 32767)
        i_vec = lax.shift_right_logical(pku, 15)
        lanew = er - er_base
        pos = R + iota
        for f in range(_D):
            val = loader(f, lanew)
            plsc.store_scatter(
                rows_v, [pos, jnp.full((_L,), f, jnp.int32)], val)
        plsc.store_scatter(bi_v, [pos], i_vec)
        R = R + _L

        @pl.when(R == _RCAP)
        def _():
            pltpu.async_copy(rows_v, out_hbm.at[bi_v], sem_out).wait()

        return jnp.where(R == _RCAP, 0, R)

    def block(b, R):
        lane0 = pl.multiple_of((c_lo + b * _CB) * 128, 128)
        descs = [
            pltpu.async_copy(
                xt_hbm.at[pl.ds(f8 * 8, 8), pl.ds(lane0, _BLK_LANES)],
                buf.at[f8], sem_in)
            for f8 in range(8)
        ]
        for d in descs:
            d.wait()

        boff = compact_block(b * _CB, b * _CB + _CB)

        def loader(f, lanew):
            return plsc.load_gather(
                buf, [jnp.full((_L,), f // 8, jnp.int32),
                      jnp.full((_L,), f % 8, jnp.int32), lanew])

        def gat_v(v, R):
            return stage_entries(v, R, boff, b * _BLK_LANES, loader)

        nv_blk = lax.shift_right_logical(boff + _L - 1, 4)
        return lax.fori_loop(0, nv_blk, gat_v, R)

    R = lax.fori_loop(0, nb, block, 0)

    pltpu.sync_copy(tail_hbm, tailb)
    tail_cr = _FULL_TCOLS - c_lo
    boff_t = compact_block(tail_cr, tail_cr + 1)

    def tail_loader(f, lanew):
        return plsc.load_gather(
            tailb, [jnp.full((_L,), f // 8, jnp.int32),
                    (f % 8) * 64 + lanew])

    def tail_v(v, R):
        return stage_entries(v, R, boff_t, tail_cr * 128, tail_loader)

    nv_tail = lax.shift_right_logical(boff_t + _L - 1, 4)
    R = lax.fori_loop(0, nv_tail, tail_v, R)

    def pad_v(p, R):
        src = jnp.full((_L,), R - 1, jnp.int32)
        pos = R + iota
        for f in range(_D):
            fv = jnp.full((_L,), f, jnp.int32)
            val = plsc.load_gather(rows_v, [src, fv])
            plsc.store_scatter(rows_v, [pos, fv], val)
        biv = plsc.load_gather(bi_v, [src])
        plsc.store_scatter(bi_v, [pos], biv)
        return R + _L

    npad = jnp.where(R > 0, lax.shift_right_logical(_RCAP - R, 4), 0)
    R = lax.fori_loop(0, npad, pad_v, R)

    @pl.when(R == _RCAP)
    def _():
        pltpu.async_copy(rows_v, out_hbm.at[bi_v], sem_out).wait()


@jax.jit
def _gather(xt, idx, tail):
    mesh = plsc.VectorSubcoreMesh(core_axis_name="c", subcore_axis_name="s")
    fn = pl.kernel(
        _gather_kernel,
        mesh=mesh,
        out_type=jax.ShapeDtypeStruct((_B, 2 * _D), jnp.float32),
        scratch_types=[
            pltpu.VMEM((_IDXCHUNK,), jnp.int32),
            pltpu.VMEM((_B,), jnp.int32),
            pltpu.VMEM((_B,), jnp.int32),
            pltpu.VMEM((8, 8, _BLK_LANES), jnp.float32),
            pltpu.VMEM((8, 512), jnp.float32),
            pltpu.VMEM((_RCAP, 2 * _D), jnp.float32),
            pltpu.VMEM((_RCAP,), jnp.int32),
            pltpu.SemaphoreType.DMA,
            pltpu.SemaphoreType.DMA,
        ],
        compiler_params=pltpu.CompilerParams(
            use_tc_tiling_on_sc=True, needs_layout_passes=False),
    )
    return fn(xt, idx, tail)[:, :_D]


def kernel(x, indices, out):
    tail = x[_TAIL_BASE:, :].T.reshape(8, 8, 64).reshape(8, 512)
    return _gather(x.T, indices, tail)


# speedup vs baseline: 3.6020x; 1.2936x over previous
import jax
import jax.numpy as jnp
from jax import lax
from jax.experimental import pallas as pl
from jax.experimental.pallas import tpu as pltpu
from jax.experimental.pallas import tpu_sc as plsc

_B = 16384
_D = 64
_NW = 32
_L = 16
_V = 1000000
_FULL_TCOLS = _V // 128
_TAIL_BASE = _FULL_TCOLS * 128
_VC = 248
_CB = 4
_BLK_LANES = _CB * 128
_RCAP = 128
_IDXCHUNK = 2048
_SELCAP = 2048


def _gather_kernel(xt_hbm, idx_hbm, tail_hbm, out_hbm,
                   idxbuf, selp, blkp, buf, tailb, rows_v, bi_v,
                   sem_a, sem_b, sem_out):
    nc = 2
    wid = lax.axis_index("s") * nc + lax.axis_index("c")
    iota = lax.iota(jnp.int32, _L)
    c_lo = wid * _VC
    c_hi_full = jnp.minimum(c_lo + _VC, _FULL_TCOLS)
    is_last = wid == (_NW - 1)
    c_hi_sel = jnp.where(is_last, _FULL_TCOLS + 1, c_hi_full)
    nb = lax.shift_right_logical(
        jnp.maximum(c_hi_full - c_lo, 0) + _CB - 1, 2)
    sems = [sem_a, sem_b]

    pltpu.sync_copy(tail_hbm, tailb)

    def issue_block(bb):
        for s in range(2):
            @pl.when(jnp.logical_and(bb < nb, (bb % 2) == s))
            def _(s=s):
                lane0 = pl.multiple_of((c_lo + bb * _CB) * 128, 128)
                for f8 in range(8):
                    pltpu.async_copy(
                        xt_hbm.at[pl.ds(f8 * 8, 8), pl.ds(lane0, _BLK_LANES)],
                        buf.at[s, f8], sems[s])

    def drain_block(bb):
        for s in range(2):
            @pl.when((bb % 2) == s)
            def _(s=s):
                for f8 in range(8):
                    pltpu.make_async_copy(
                        xt_hbm.at[pl.ds(f8 * 8, 8), pl.ds(0, _BLK_LANES)],
                        buf.at[s, f8], sems[s]).wait()

    def run_pass(p, R):
        issue_block(jnp.int32(0))

        def sel_chunk(q, carry):
            pltpu.sync_copy(
                idx_hbm.at[pl.ds(q * _IDXCHUNK, _IDXCHUNK)], idxbuf)

            def sel_v(v, carry):
                tot, off = carry
                e = idxbuf[pl.ds(v * _L, _L)]
                c = lax.shift_right_logical(e, 7)
                m = jnp.logical_and(c >= c_lo, c < c_hi_sel)
                mi = m.astype(jnp.int32)
                ordl = tot + plsc.cumsum(mi) - mi
                m2 = jnp.logical_and(
                    m, jnp.logical_and(ordl >= p * _SELCAP,
                                       ordl < (p + 1) * _SELCAP))
                i_g = q * _IDXCHUNK + v * _L + iota
                pk = jnp.bitwise_or(lax.shift_left(i_g, 15), e - c_lo * 128)
                plsc.store_compressed(selp.at[pl.ds(off, _L)], pk, mask=m2)
                return (tot + jnp.sum(mi), off + jnp.sum(m2.astype(jnp.int32)))

            return lax.fori_loop(0, _IDXCHUNK // _L, sel_v, carry)

        tot, nsel = lax.fori_loop(0, _B // _IDXCHUNK, sel_chunk, (0, 0))
        nv_sel = lax.shift_right_logical(nsel + _L - 1, 4)

        def compact_block(lo_r, hi_r):
            def cmp_v(v, boff):
                pk = selp[pl.ds(v * _L, _L)]
                valid = (v * _L + iota) < nsel
                cr = lax.shift_right_logical(jnp.bitwise_and(pk, 32767), 7)
                m = jnp.logical_and(valid,
                                    jnp.logical_and(cr >= lo_r, cr < hi_r))
                plsc.store_compressed(blkp.at[pl.ds(boff, _L)], pk, mask=m)
                return boff + jnp.sum(m.astype(jnp.int32))

            return lax.fori_loop(0, nv_sel, cmp_v, 0)

        def stage_entries(v, R, boff, er_base, loader):
            pk = blkp[pl.ds(v * _L, _L)]
            valid = (v * _L + iota) < boff
            pkm = jnp.where(valid, pk, jnp.int32(1 << 30))
            mn = jnp.min(pkm, axis=0)
            pku = jnp.where(valid, pk, jnp.full((_L,), 1, jnp.int32) * mn)
            er = jnp.bitwise_and(pku, 32767)
            i_vec = lax.shift_right_logical(pku, 15)
            lanew = er - er_base
            pos = R + iota
            for f in range(_D):
                val = loader(f, lanew)
                plsc.store_scatter(
                    rows_v, [pos, jnp.full((_L,), f, jnp.int32)], val)
            plsc.store_scatter(bi_v, [pos], i_vec)
            R = R + _L

            @pl.when(R == _RCAP)
            def _():
                pltpu.async_copy(rows_v, out_hbm.at[bi_v], sem_out).wait()

            return jnp.where(R == _RCAP, 0, R)

        def block(b, R):
            issue_block(b + 1)
            drain_block(b)
            boff = compact_block(b * _CB, b * _CB + _CB)
            par = b % 2

            def loader(f, lanew):
                return plsc.load_gather(
                    buf, [jnp.full((_L,), 1, jnp.int32) * par,
                          jnp.full((_L,), f // 8, jnp.int32),
                          jnp.full((_L,), f % 8, jnp.int32), lanew])

            def gat_v(v, R):
                return stage_entries(v, R, boff, b * _BLK_LANES, loader)

            nv_blk = lax.shift_right_logical(boff + _L - 1, 4)
            return lax.fori_loop(0, nv_blk, gat_v, R)

        R = lax.fori_loop(0, nb, block, R)

        tail_cr = _FULL_TCOLS - c_lo
        boff_t = compact_block(tail_cr, tail_cr + 1)

        def tail_loader(f, lanew):
            return plsc.load_gather(
                tailb, [jnp.full((_L,), f // 8, jnp.int32),
                        (f % 8) * 64 + lanew])

        def tail_v(v, R):
            return stage_entries(v, R, boff_t, tail_cr * 128, tail_loader)

        nv_tail = lax.shift_right_logical(boff_t + _L - 1, 4)
        R = lax.fori_loop(0, nv_tail, tail_v, R)
        return tot, R

    def cond(carry):
        p, R, tot = carry
        return p * _SELCAP < tot

    def body(carry):
        p, R, _ = carry
        tot, R = run_pass(p, R)
        return (p + 1, R, tot)

    _, R, _ = lax.while_loop(cond, body, (0, 0, 1))

    def pad_v(_, R):
        src = jnp.full((_L,), R - 1, jnp.int32)
        pos = R + iota
        for f in range(_D):
            fv = jnp.full((_L,), f, jnp.int32)
            val = plsc.load_gather(rows_v, [src, fv])
            plsc.store_scatter(rows_v, [pos, fv], val)
        biv = plsc.load_gather(bi_v, [src])
        plsc.store_scatter(bi_v, [pos], biv)
        return R + _L

    npad = jnp.where(R > 0, lax.shift_right_logical(_RCAP - R, 4), 0)
    R = lax.fori_loop(0, npad, pad_v, R)

    @pl.when(R == _RCAP)
    def _():
        pltpu.async_copy(rows_v, out_hbm.at[bi_v], sem_out).wait()


@jax.jit
def _gather(xt, idx, tail):
    mesh = plsc.VectorSubcoreMesh(core_axis_name="c", subcore_axis_name="s")
    fn = pl.kernel(
        _gather_kernel,
        mesh=mesh,
        out_type=jax.ShapeDtypeStruct((_B, 2 * _D), jnp.float32),
        scratch_types=[
            pltpu.VMEM((_IDXCHUNK,), jnp.int32),
            pltpu.VMEM((_SELCAP,), jnp.int32),
            pltpu.VMEM((_SELCAP,), jnp.int32),
            pltpu.VMEM((2, 8, 8, _BLK_LANES), jnp.float32),
            pltpu.VMEM((8, 512), jnp.float32),
            pltpu.VMEM((_RCAP, 2 * _D), jnp.float32),
            pltpu.VMEM((_RCAP,), jnp.int32),
            pltpu.SemaphoreType.DMA,
            pltpu.SemaphoreType.DMA,
            pltpu.SemaphoreType.DMA,
        ],
        compiler_params=pltpu.CompilerParams(
            use_tc_tiling_on_sc=True, needs_layout_passes=False),
    )
    return fn(xt, idx, tail)[:, :_D]


def kernel(x, indices, out):
    tail = x[_TAIL_BASE:, :].T.reshape(8, 8, 64).reshape(8, 512)
    return _gather(x.T, indices, tail)


# speedup vs baseline: 3.6172x; 1.0042x over previous
import jax
import jax.numpy as jnp
from jax import lax
from jax.experimental import pallas as pl
from jax.experimental.pallas import tpu as pltpu
from jax.experimental.pallas import tpu_sc as plsc

_B = 16384
_D = 64
_NW = 32
_L = 16
_V = 1000000
_FULL_TCOLS = _V // 128
_TAIL_BASE = _FULL_TCOLS * 128
_VC = 248
_CB = 4
_BLK_LANES = _CB * 128
_RCAP = 128
_IDXCHUNK = 2048
_SELCAP = 2048


def _gather_kernel(xt_hbm, idx_hbm, tail_hbm, out_hbm,
                   idxbuf, selp, blkp, buf, tailb, rows_v, bi_v,
                   sem_a, sem_b, sem_out):
    nc = 2
    wid = lax.axis_index("s") * nc + lax.axis_index("c")
    iota = lax.iota(jnp.int32, _L)
    c_lo = wid * _VC
    c_hi_full = jnp.minimum(c_lo + _VC, _FULL_TCOLS)
    is_last = wid == (_NW - 1)
    c_hi_sel = jnp.where(is_last, _FULL_TCOLS + 1, c_hi_full)
    nb = lax.shift_right_logical(
        jnp.maximum(c_hi_full - c_lo, 0) + _CB - 1, 2)
    sems = [sem_a, sem_b]

    pltpu.sync_copy(tail_hbm, tailb)

    def issue_block(bb):
        for s in range(2):
            @pl.when(jnp.logical_and(bb < nb, (bb % 2) == s))
            def _(s=s):
                lane0 = pl.multiple_of((c_lo + bb * _CB) * 128, 128)
                pltpu.async_copy(
                    xt_hbm.at[:, pl.ds(lane0, _BLK_LANES)],
                    buf.at[s], sems[s])

    def drain_block(bb):
        for s in range(2):
            @pl.when((bb % 2) == s)
            def _(s=s):
                pltpu.make_async_copy(
                    xt_hbm.at[:, pl.ds(0, _BLK_LANES)],
                    buf.at[s], sems[s]).wait()

    def run_pass(p, R):
        issue_block(jnp.int32(0))

        def sel_chunk(q, carry):
            pltpu.sync_copy(
                idx_hbm.at[pl.ds(q * _IDXCHUNK, _IDXCHUNK)], idxbuf)

            def sel_v(v, carry):
                tot, off = carry
                e = idxbuf[pl.ds(v * _L, _L)]
                c = lax.shift_right_logical(e, 7)
                m = jnp.logical_and(c >= c_lo, c < c_hi_sel)
                mi = m.astype(jnp.int32)
                ordl = tot + plsc.cumsum(mi) - mi
                m2 = jnp.logical_and(
                    m, jnp.logical_and(ordl >= p * _SELCAP,
                                       ordl < (p + 1) * _SELCAP))
                i_g = q * _IDXCHUNK + v * _L + iota
                pk = jnp.bitwise_or(lax.shift_left(i_g, 15), e - c_lo * 128)
                plsc.store_compressed(selp.at[pl.ds(off, _L)], pk, mask=m2)
                return (tot + jnp.sum(mi), off + jnp.sum(m2.astype(jnp.int32)))

            return lax.fori_loop(0, _IDXCHUNK // _L, sel_v, carry)

        tot, nsel = lax.fori_loop(0, _B // _IDXCHUNK, sel_chunk, (0, 0))
        nv_sel = lax.shift_right_logical(nsel + _L - 1, 4)

        def compact_block(lo_r, hi_r):
            def cmp_v(v, boff):
                pk = selp[pl.ds(v * _L, _L)]
                valid = (v * _L + iota) < nsel
                cr = lax.shift_right_logical(jnp.bitwise_and(pk, 32767), 7)
                m = jnp.logical_and(valid,
                                    jnp.logical_and(cr >= lo_r, cr < hi_r))
                plsc.store_compressed(blkp.at[pl.ds(boff, _L)], pk, mask=m)
                return boff + jnp.sum(m.astype(jnp.int32))

            return lax.fori_loop(0, nv_sel, cmp_v, 0)

        def stage_entries(v, R, boff, er_base, loader):
            pk = blkp[pl.ds(v * _L, _L)]
            valid = (v * _L + iota) < boff
            pkm = jnp.where(valid, pk, jnp.int32(1 << 30))
            mn = jnp.min(pkm, axis=0)
            pku = jnp.where(valid, pk, jnp.full((_L,), 1, jnp.int32) * mn)
            er = jnp.bitwise_and(pku, 32767)
            i_vec = lax.shift_right_logical(pku, 15)
            lanew = er - er_base
            pos = R + iota
            for f in range(_D):
                val = loader(f, lanew)
                plsc.store_scatter(
                    rows_v, [pos, jnp.full((_L,), f, jnp.int32)], val)
            plsc.store_scatter(bi_v, [pos], i_vec)
            R = R + _L

            @pl.when(R == _RCAP)
            def _():
                pltpu.async_copy(rows_v, out_hbm.at[bi_v], sem_out).wait()

            return jnp.where(R == _RCAP, 0, R)

        def block(b, R):
            issue_block(b + 1)
            drain_block(b)
            boff = compact_block(b * _CB, b * _CB + _CB)
            par = b % 2

            def loader(f, lanew):
                return plsc.load_gather(
                    buf, [jnp.full((_L,), 1, jnp.int32) * par,
                          jnp.full((_L,), f, jnp.int32), lanew])

            def gat_v(v, R):
                return stage_entries(v, R, boff, b * _BLK_LANES, loader)

            nv_blk = lax.shift_right_logical(boff + _L - 1, 4)
            return lax.fori_loop(0, nv_blk, gat_v, R)

        R = lax.fori_loop(0, nb, block, R)

        tail_cr = _FULL_TCOLS - c_lo
        boff_t = compact_block(tail_cr, tail_cr + 1)

        def tail_loader(f, lanew):
            return plsc.load_gather(
                tailb, [jnp.full((_L,), f // 8, jnp.int32),
                        (f % 8) * 64 + lanew])

        def tail_v(v, R):
            return stage_entries(v, R, boff_t, tail_cr * 128, tail_loader)

        nv_tail = lax.shift_right_logical(boff_t + _L - 1, 4)
        R = lax.fori_loop(0, nv_tail, tail_v, R)
        return tot, R

    def cond(carry):
        p, R, tot = carry
        return p * _SELCAP < tot

    def body(carry):
        p, R, _ = carry
        tot, R = run_pass(p, R)
        return (p + 1, R, tot)

    _, R, _ = lax.while_loop(cond, body, (0, 0, 1))

    def pad_v(_, R):
        src = jnp.full((_L,), R - 1, jnp.int32)
        pos = R + iota
        for f in range(_D):
            fv = jnp.full((_L,), f, jnp.int32)
            val = plsc.load_gather(rows_v, [src, fv])
            plsc.store_scatter(rows_v, [pos, fv], val)
        biv = plsc.load_gather(bi_v, [src])
        plsc.store_scatter(bi_v, [pos], biv)
        return R + _L

    npad = jnp.where(R > 0, lax.shift_right_logical(_RCAP - R, 4), 0)
    R = lax.fori_loop(0, npad, pad_v, R)

    @pl.when(R == _RCAP)
    def _():
        pltpu.async_copy(rows_v, out_hbm.at[bi_v], sem_out).wait()


@jax.jit
def _gather(xt, idx, tail):
    mesh = plsc.VectorSubcoreMesh(core_axis_name="c", subcore_axis_name="s")
    fn = pl.kernel(
        _gather_kernel,
        mesh=mesh,
        out_type=jax.ShapeDtypeStruct((_B, 2 * _D), jnp.float32),
        scratch_types=[
            pltpu.VMEM((_IDXCHUNK,), jnp.int32),
            pltpu.VMEM((_SELCAP,), jnp.int32),
            pltpu.VMEM((_SELCAP,), jnp.int32),
            pltpu.VMEM((2, 64, _BLK_LANES), jnp.float32),
            pltpu.VMEM((8, 512), jnp.float32),
            pltpu.VMEM((_RCAP, 2 * _D), jnp.float32),
            pltpu.VMEM((_RCAP,), jnp.int32),
            pltpu.SemaphoreType.DMA,
            pltpu.SemaphoreType.DMA,
            pltpu.SemaphoreType.DMA,
        ],
        compiler_params=pltpu.CompilerParams(
            use_tc_tiling_on_sc=True, needs_layout_passes=False),
    )
    return fn(xt, idx, tail)[:, :_D]


def kernel(x, indices, out):
    tail = x[_TAIL_BASE:, :].T.reshape(8, 8, 64).reshape(8, 512)
    return _gather(x.T, indices, tail)


# speedup vs baseline: 3.6638x; 1.0129x over previous
import jax
import jax.numpy as jnp
from jax import lax
from jax.experimental import pallas as pl
from jax.experimental.pallas import tpu as pltpu
from jax.experimental.pallas import tpu_sc as plsc

_B = 16384
_D = 64
_NW = 32
_L = 16
_V = 1000000
_FULL_TCOLS = _V // 128
_TAIL_BASE = _FULL_TCOLS * 128
_VC = 248
_CB = 4
_BLK_LANES = _CB * 128
_RCAP = 128
_IDXCHUNK = 2048
_SELCAP = 2048


def _gather_kernel(xt_hbm, idx_hbm, tail_hbm, out_hbm,
                   idxbuf, selp, blkp, buf, tailb, rows_v, bi2_v,
                   sem_a, sem_b, sem_out):
    nc = 2
    wid = lax.axis_index("s") * nc + lax.axis_index("c")
    iota = lax.iota(jnp.int32, _L)
    c_lo = wid * _VC
    c_hi_full = jnp.minimum(c_lo + _VC, _FULL_TCOLS)
    is_last = wid == (_NW - 1)
    c_hi_sel = jnp.where(is_last, _FULL_TCOLS + 1, c_hi_full)
    nb = lax.shift_right_logical(
        jnp.maximum(c_hi_full - c_lo, 0) + _CB - 1, 2)
    sems = [sem_a, sem_b]

    pltpu.sync_copy(tail_hbm, tailb)

    def issue_block(bb):
        for s in range(2):
            @pl.when(jnp.logical_and(bb < nb, (bb % 2) == s))
            def _(s=s):
                lane0 = pl.multiple_of((c_lo + bb * _CB) * 128, 128)
                pltpu.async_copy(
                    xt_hbm.at[:, pl.ds(lane0, _BLK_LANES)],
                    buf.at[s], sems[s])

    def drain_block(bb):
        for s in range(2):
            @pl.when((bb % 2) == s)
            def _(s=s):
                pltpu.make_async_copy(
                    xt_hbm.at[:, pl.ds(0, _BLK_LANES)],
                    buf.at[s], sems[s]).wait()

    def run_pass(p, R):
        issue_block(jnp.int32(0))

        def sel_chunk(q, carry):
            pltpu.sync_copy(
                idx_hbm.at[pl.ds(q * _IDXCHUNK, _IDXCHUNK)], idxbuf)

            def sel_v(v, carry):
                tot, off = carry
                e = idxbuf[pl.ds(v * _L, _L)]
                c = lax.shift_right_logical(e, 7)
                m = jnp.logical_and(c >= c_lo, c < c_hi_sel)
                mi = m.astype(jnp.int32)
                ordl = tot + plsc.cumsum(mi) - mi
                m2 = jnp.logical_and(
                    m, jnp.logical_and(ordl >= p * _SELCAP,
                                       ordl < (p + 1) * _SELCAP))
                i_g = q * _IDXCHUNK + v * _L + iota
                pk = jnp.bitwise_or(lax.shift_left(i_g, 15), e - c_lo * 128)
                plsc.store_compressed(selp.at[pl.ds(off, _L)], pk, mask=m2)
                return (tot + jnp.sum(mi), off + jnp.sum(m2.astype(jnp.int32)))

            return lax.fori_loop(0, _IDXCHUNK // _L, sel_v, carry)

        tot, nsel = lax.fori_loop(0, _B // _IDXCHUNK, sel_chunk, (0, 0))
        nv_sel = lax.shift_right_logical(nsel + _L - 1, 4)

        def compact_block(lo_r, hi_r):
            def cmp_v(v, boff):
                pk = selp[pl.ds(v * _L, _L)]
                valid = (v * _L + iota) < nsel
                cr = lax.shift_right_logical(jnp.bitwise_and(pk, 32767), 7)
                m = jnp.logical_and(valid,
                                    jnp.logical_and(cr >= lo_r, cr < hi_r))
                plsc.store_compressed(blkp.at[pl.ds(boff, _L)], pk, mask=m)
                return boff + jnp.sum(m.astype(jnp.int32))

            return lax.fori_loop(0, nv_sel, cmp_v, 0)

        def stage_entries(v, R, boff, er_base, loader):
            pk = blkp[pl.ds(v * _L, _L)]
            valid = (v * _L + iota) < boff
            pkm = jnp.where(valid, pk, jnp.int32(1 << 30))
            mn = jnp.min(pkm, axis=0)
            pku = jnp.where(valid, pk, jnp.full((_L,), 1, jnp.int32) * mn)
            er = jnp.bitwise_and(pku, 32767)
            i_vec = lax.shift_right_logical(pku, 15)
            lanew = er - er_base
            pos = R + iota
            for f in range(_D):
                val = loader(f, lanew)
                plsc.store_scatter(
                    rows_v, [pos, jnp.full((_L,), f, jnp.int32)], val)
            plsc.store_scatter(
                bi2_v, [lax.shift_right_logical(pos, 7),
                        jnp.bitwise_and(pos, 127)], i_vec)
            R = R + jnp.minimum(boff - v * _L, _L)

            @pl.when(R >= _RCAP)
            def _():
                pltpu.async_copy(
                    rows_v.at[pl.ds(0, _RCAP)], out_hbm.at[bi2_v.at[0]],
                    sem_out).wait()
                src = _RCAP + iota
                for f in range(_D):
                    fv = jnp.full((_L,), f, jnp.int32)
                    ovf = plsc.load_gather(rows_v, [src, fv])
                    plsc.store_scatter(rows_v, [iota, fv], ovf)
                bovf = plsc.load_gather(
                    bi2_v, [lax.shift_right_logical(src, 7),
                            jnp.bitwise_and(src, 127)])
                plsc.store_scatter(
                    bi2_v, [jnp.full((_L,), 0, jnp.int32), iota], bovf)

            return jnp.where(R >= _RCAP, R - _RCAP, R)

        def block(b, R):
            issue_block(b + 1)
            drain_block(b)
            boff = compact_block(b * _CB, b * _CB + _CB)
            par = b % 2

            def loader(f, lanew):
                return plsc.load_gather(
                    buf, [jnp.full((_L,), 1, jnp.int32) * par,
                          jnp.full((_L,), f, jnp.int32), lanew])

            def gat_v(v, R):
                return stage_entries(v, R, boff, b * _BLK_LANES, loader)

            nv_blk = lax.shift_right_logical(boff + _L - 1, 4)
            return lax.fori_loop(0, nv_blk, gat_v, R)

        R = lax.fori_loop(0, nb, block, R)

        tail_cr = _FULL_TCOLS - c_lo
        boff_t = compact_block(tail_cr, tail_cr + 1)

        def tail_loader(f, lanew):
            return plsc.load_gather(
                tailb, [jnp.full((_L,), f // 8, jnp.int32),
                        (f % 8) * 64 + lanew])

        def tail_v(v, R):
            return stage_entries(v, R, boff_t, tail_cr * 128, tail_loader)

        nv_tail = lax.shift_right_logical(boff_t + _L - 1, 4)
        R = lax.fori_loop(0, nv_tail, tail_v, R)
        return tot, R

    def cond(carry):
        p, R, tot = carry
        return p * _SELCAP < tot

    def body(carry):
        p, R, _ = carry
        tot, R = run_pass(p, R)
        return (p + 1, R, tot)

    _, R, _ = lax.while_loop(cond, body, (0, 0, 1))

    def pad_v(_, R):
        src = jnp.full((_L,), R - 1, jnp.int32)
        pos = R + iota
        for f in range(_D):
            fv = jnp.full((_L,), f, jnp.int32)
            val = plsc.load_gather(rows_v, [src, fv])
            plsc.store_scatter(rows_v, [pos, fv], val)
        biv = plsc.load_gather(
            bi2_v, [lax.shift_right_logical(src, 7),
                    jnp.bitwise_and(src, 127)])
        plsc.store_scatter(
            bi2_v, [lax.shift_right_logical(pos, 7),
                    jnp.bitwise_and(pos, 127)], biv)
        return R + _L

    npad = jnp.where(
        R > 0, lax.shift_right_logical(_RCAP - R + _L - 1, 4), 0)
    R = lax.fori_loop(0, npad, pad_v, R)

    @pl.when(R >= _RCAP)
    def _():
        pltpu.async_copy(
            rows_v.at[pl.ds(0, _RCAP)], out_hbm.at[bi2_v.at[0]],
            sem_out).wait()


@jax.jit
def _gather(xt, idx, tail):
    mesh = plsc.VectorSubcoreMesh(core_axis_name="c", subcore_axis_name="s")
    fn = pl.kernel(
        _gather_kernel,
        mesh=mesh,
        out_type=jax.ShapeDtypeStruct((_B, 2 * _D), jnp.float32),
        scratch_types=[
            pltpu.VMEM((_IDXCHUNK,), jnp.int32),
            pltpu.VMEM((_SELCAP,), jnp.int32),
            pltpu.VMEM((_SELCAP,), jnp.int32),
            pltpu.VMEM((2, 64, _BLK_LANES), jnp.float32),
            pltpu.VMEM((8, 512), jnp.float32),
            pltpu.VMEM((_RCAP + _L, 2 * _D), jnp.float32),
            pltpu.VMEM((2, _RCAP), jnp.int32),
            pltpu.SemaphoreType.DMA,
            pltpu.SemaphoreType.DMA,
            pltpu.SemaphoreType.DMA,
        ],
        compiler_params=pltpu.CompilerParams(
            use_tc_tiling_on_sc=True, needs_layout_passes=False),
    )
    return fn(xt, idx, tail)[:, :_D]


def kernel(x, indices, out):
    tail = x[_TAIL_BASE:, :].T.reshape(8, 8, 64).reshape(8, 512)
    return _gather(x.T, indices, tail)
